# baseline (speedup 1.0000x reference)
import functools
import math

import jax
import jax.numpy as jnp
from jax import lax
from jax.experimental import pallas as pl
from jax.experimental.pallas import tpu as pltpu
from jax.experimental.pallas import tpu_sc as plsc

NUM_NODES = 100000
SIZE = 10
HID = 64
TDIM = 64
DIMS = HID + TDIM
B = 16384

NW = 32
CH = 64
EPW = B // NW
NCHUNK = EPW // CH


def _sc_gather(nid2d, msg2d, t_flat, msg_count):
  mesh = plsc.VectorSubcoreMesh(core_axis_name="c", subcore_axis_name="s")

  @functools.partial(
      pl.kernel,
      mesh=mesh,
      compiler_params=pltpu.CompilerParams(use_tc_tiling_on_sc=False),
      out_type=[
          jax.ShapeDtypeStruct((B, SIZE * HID), jnp.float32),
          jax.ShapeDtypeStruct((NW, SIZE, EPW), jnp.float32),
          jax.ShapeDtypeStruct((B,), jnp.int32),
      ],
      scratch_types=[
          pltpu.VMEM((NCHUNK, CH), jnp.int32),
          pltpu.VMEM((CH, SIZE * HID), jnp.float32),
          pltpu.VMEM((CH, SIZE * HID), jnp.float32),
          pltpu.VMEM((SIZE, EPW), jnp.float32),
          pltpu.VMEM((NCHUNK * SIZE, CH), jnp.int32),
          pltpu.VMEM((EPW,), jnp.int32),
          pltpu.SemaphoreType.DMA,
          pltpu.SemaphoreType.DMA,
          pltpu.SemaphoreType.DMA,
      ],
  )
  def gather_kernel(nid_hbm, msg_hbm, t_hbm, mc_hbm,
                    out_msg, out_t, out_mc,
                    idx_v, buf_a, buf_b, t_v, tidx_v, mc_v,
                    sem_a, sem_b, sem_small):
    wid = lax.axis_index("s") * 2 + lax.axis_index("c")
    rbase = wid * NCHUNK
    ebase = wid * EPW

    pltpu.sync_copy(nid_hbm.at[pl.ds(rbase, NCHUNK)], idx_v)

    handles = []
    for j in range(NCHUNK):
      for g in range(CH // 16):
        v = idx_v[j, pl.ds(g * 16, 16)] * SIZE
        for s in range(SIZE):
          tidx_v[j * SIZE + s, pl.ds(g * 16, 16)] = v + s
      for s in range(SIZE):
        handles.append(pltpu.async_copy(
            t_hbm.at[tidx_v.at[j * SIZE + s]],
            t_v.at[s, pl.ds(j * CH, CH)], sem_small))
      handles.append(pltpu.async_copy(
          mc_hbm.at[idx_v.at[j]], mc_v.at[pl.ds(j * CH, CH)], sem_small))
    for h in handles:
      h.wait()
    pltpu.sync_copy(t_v, out_t.at[wid])
    pltpu.sync_copy(mc_v, out_mc.at[pl.ds(ebase, EPW)])

    bufs = (buf_a, buf_b)
    sems = (sem_a, sem_b)
    prev = pltpu.async_copy(msg_hbm.at[idx_v.at[0]], bufs[0], sems[0])
    for j in range(1, NCHUNK):
      cur = pltpu.async_copy(msg_hbm.at[idx_v.at[j]], bufs[j % 2],
                             sems[j % 2])
      prev.wait()
      pltpu.sync_copy(bufs[(j - 1) % 2],
                      out_msg.at[pl.ds(ebase + (j - 1) * CH, CH)])
      prev = cur
    prev.wait()
    pltpu.sync_copy(bufs[(NCHUNK - 1) % 2],
                    out_msg.at[pl.ds(ebase + (NCHUNK - 1) * CH, CH)])

  return gather_kernel(nid2d, msg2d, t_flat, msg_count)


BB = 256
RR = BB * SIZE


def _tc_mixer(msg_rows, t_rows, tref_rows, mc_rows, freq_row,
              tW_t, tb, cW_t, cb, tg, tbeta, cg, cbeta):
  grid = (B // BB,)

  def body(msg_ref, t_ref, tr_ref, mc_ref, freq_ref,
           tw_ref, tb_ref, cw_ref, cb_ref,
           tg_ref, tbt_ref, cg_ref, cbt_ref, out_ref):
    dt = tr_ref[...] - t_ref[...]
    enc = jnp.cos(dt * freq_ref[...]) * (1.0 / math.sqrt(TDIM))
    slot = lax.broadcasted_iota(jnp.int32, (RR, 1), 0) % SIZE
    mask = (slot.astype(jnp.float32) < mc_ref[...]).astype(jnp.float32)
    x = jnp.concatenate([enc, msg_ref[...]], axis=1) * mask

    def ln(v, g, b):
      mu = jnp.mean(v, axis=1, keepdims=True)
      var = jnp.mean((v - mu) ** 2, axis=1, keepdims=True)
      return (v - mu) * lax.rsqrt(var + 1e-5) * g + b

    def gelu(v):
      return 0.5 * v * (1.0 + lax.erf(v * (1.0 / math.sqrt(2.0))))

    h = ln(x, tg_ref[...], tbt_ref[...])
    h = gelu(jnp.dot(h, tw_ref[...], preferred_element_type=jnp.float32)
             + tb_ref[...])
    x = x + h
    h = ln(x, cg_ref[...], cbt_ref[...])
    h = gelu(jnp.dot(h, cw_ref[...], preferred_element_type=jnp.float32)
             + cb_ref[...])
    x = x + h

    rowi = lax.broadcasted_iota(jnp.int32, (BB, RR), 1) // SIZE
    bi = lax.broadcasted_iota(jnp.int32, (BB, RR), 0)
    sel = jnp.where(rowi == bi, 1.0 / SIZE, 0.0)
    out_ref[...] = jnp.dot(sel, x, preferred_element_type=jnp.float32)

  col = pl.BlockSpec((RR, 1), lambda i: (i, 0))
  full = lambda shp: pl.BlockSpec(shp, lambda i: (0, 0))
  return pl.pallas_call(
      body,
      grid=grid,
      in_specs=[
          pl.BlockSpec((RR, HID), lambda i: (i, 0)),
          col, col, col,
          full((1, TDIM)),
          full((DIMS, DIMS)), full((1, DIMS)),
          full((DIMS, DIMS)), full((1, DIMS)),
          full((1, DIMS)), full((1, DIMS)), full((1, DIMS)), full((1, DIMS)),
      ],
      out_specs=pl.BlockSpec((BB, DIMS), lambda i: (i, 0)),
      out_shape=jax.ShapeDtypeStruct((B, DIMS), jnp.float32),
  )(msg_rows, t_rows, tref_rows, mc_rows, freq_row,
    tW_t, tb, cW_t, cb, tg, tbeta, cg, cbeta)


def kernel(n_id, t_ref, msg_store, t_store, msg_count,
           token_gamma, token_beta, token_W, token_b,
           chan_gamma, chan_beta, chan_W, chan_b):
  nid2d = n_id.astype(jnp.int32).reshape(B // CH, CH)
  msg2d = msg_store.reshape(NUM_NODES, SIZE * HID)

  msg_g, t_g, mc_g = _sc_gather(nid2d, msg2d, t_store.reshape(-1), msg_count)

  msg_rows = msg_g.reshape(B * SIZE, HID)
  t_rows = jnp.transpose(t_g, (0, 2, 1)).reshape(B * SIZE, 1)
  tref_rows = jnp.repeat(t_ref, SIZE).reshape(B * SIZE, 1)
  mc_rows = jnp.repeat(mc_g.astype(jnp.float32), SIZE).reshape(B * SIZE, 1)

  freq_row = (1.0 / (10.0 ** jnp.linspace(0.0, 9.0, TDIM,
                                          dtype=jnp.float32))).reshape(1, TDIM)

  return _tc_mixer(msg_rows, t_rows, tref_rows, mc_rows, freq_row,
                   token_W.T, token_b.reshape(1, DIMS),
                   chan_W.T, chan_b.reshape(1, DIMS),
                   token_gamma.reshape(1, DIMS), token_beta.reshape(1, DIMS),
                   chan_gamma.reshape(1, DIMS), chan_beta.reshape(1, DIMS))


# speedup vs baseline: 1.2387x; 1.2387x over previous
import functools
import math

import jax
import jax.numpy as jnp
from jax import lax
from jax.experimental import pallas as pl
from jax.experimental.pallas import tpu as pltpu
from jax.experimental.pallas import tpu_sc as plsc

NUM_NODES = 100000
SIZE = 10
HID = 64
TDIM = 64
DIMS = HID + TDIM
B = 16384

NW = 32
CH = 64
EPW = B // NW
NCHUNK = EPW // CH


def _sc_gather(nid2d, msg2d, t_flat, msg_count):
  mesh = plsc.VectorSubcoreMesh(core_axis_name="c", subcore_axis_name="s")

  @functools.partial(
      pl.kernel,
      mesh=mesh,
      compiler_params=pltpu.CompilerParams(use_tc_tiling_on_sc=False),
      out_type=[
          jax.ShapeDtypeStruct((B, SIZE * HID), jnp.float32),
          jax.ShapeDtypeStruct((NW, SIZE, EPW), jnp.float32),
          jax.ShapeDtypeStruct((B,), jnp.int32),
      ],
      scratch_types=[
          pltpu.VMEM((NCHUNK, CH), jnp.int32),
          pltpu.VMEM((CH, SIZE * HID), jnp.float32),
          pltpu.VMEM((CH, SIZE * HID), jnp.float32),
          pltpu.VMEM((SIZE, EPW), jnp.float32),
          pltpu.VMEM((NCHUNK * SIZE, CH), jnp.int32),
          pltpu.VMEM((EPW,), jnp.int32),
          pltpu.SemaphoreType.DMA,
          pltpu.SemaphoreType.DMA,
          pltpu.SemaphoreType.DMA,
      ],
  )
  def gather_kernel(nid_hbm, msg_hbm, t_hbm, mc_hbm,
                    out_msg, out_t, out_mc,
                    idx_v, buf_a, buf_b, t_v, tidx_v, mc_v,
                    sem_a, sem_b, sem_small):
    wid = lax.axis_index("s") * 2 + lax.axis_index("c")
    rbase = wid * NCHUNK
    ebase = wid * EPW

    pltpu.sync_copy(nid_hbm.at[pl.ds(rbase, NCHUNK)], idx_v)

    handles = []
    for j in range(NCHUNK):
      for g in range(CH // 16):
        v = idx_v[j, pl.ds(g * 16, 16)] * SIZE
        for s in range(SIZE):
          tidx_v[j * SIZE + s, pl.ds(g * 16, 16)] = v + s
      for s in range(SIZE):
        handles.append(pltpu.async_copy(
            t_hbm.at[tidx_v.at[j * SIZE + s]],
            t_v.at[s, pl.ds(j * CH, CH)], sem_small))
      handles.append(pltpu.async_copy(
          mc_hbm.at[idx_v.at[j]], mc_v.at[pl.ds(j * CH, CH)], sem_small))
    for h in handles:
      h.wait()
    pltpu.sync_copy(t_v, out_t.at[wid])
    pltpu.sync_copy(mc_v, out_mc.at[pl.ds(ebase, EPW)])

    bufs = (buf_a, buf_b)
    sems = (sem_a, sem_b)
    prev = pltpu.async_copy(msg_hbm.at[idx_v.at[0]], bufs[0], sems[0])
    for j in range(1, NCHUNK):
      cur = pltpu.async_copy(msg_hbm.at[idx_v.at[j]], bufs[j % 2],
                             sems[j % 2])
      prev.wait()
      pltpu.sync_copy(bufs[(j - 1) % 2],
                      out_msg.at[pl.ds(ebase + (j - 1) * CH, CH)])
      prev = cur
    prev.wait()
    pltpu.sync_copy(bufs[(NCHUNK - 1) % 2],
                    out_msg.at[pl.ds(ebase + (NCHUNK - 1) * CH, CH)])

  return gather_kernel(nid2d, msg2d, t_flat, msg_count)


BB = 256
RR = BB * SIZE

_COS_C = [0.125 * c for c in
          (1.0, -19.739208, 64.93939, -85.45669, 60.242466,
           -26.406763, 7.8066154, -1.4609568)]
_INV_2PI = 1.0 / (2.0 * math.pi)
_MAGIC = 1.5 * 2.0 ** 23

import numpy as np

_SLOT_COL = np.tile(np.arange(SIZE, dtype=np.float32), BB).reshape(BB * SIZE, 1)
_SEL = (np.repeat(np.eye(BB, dtype=np.float32), SIZE, axis=1) / SIZE)


def _tc_mixer(msg_rows, t_rows, tref_rows, mc_rows, freq_row,
              tW_t, tb, cW_t, cb, tg, tbeta, cg, cbeta):
  grid = (B // BB,)

  def body(msg_ref, t_ref, tr_ref, mc_ref, freq_ref, slot_ref, sel_ref,
           tw_ref, tb_ref, cw_ref, cb_ref,
           tg_ref, tbt_ref, cg_ref, cbt_ref, out_ref):
    dt = tr_ref[...] - t_ref[...]
    y = (dt * freq_ref[...]) * _INV_2PI
    y = y - lax.round(y, lax.RoundingMethod.TO_NEAREST_EVEN)
    u = y * y
    enc = _COS_C[7]
    for k in range(6, -1, -1):
      enc = enc * u + _COS_C[k]
    mask = (slot_ref[...] < mc_ref[...]).astype(jnp.float32)
    x = jnp.concatenate([enc, msg_ref[...]], axis=1) * mask

    def ln(v, g, b):
      mu = jnp.mean(v, axis=1, keepdims=True)
      var = jnp.mean((v - mu) ** 2, axis=1, keepdims=True)
      return (v - mu) * lax.rsqrt(var + 1e-5) * g + b

    def gelu(v):
      return 0.5 * v * (1.0 + lax.erf(v * (1.0 / math.sqrt(2.0))))

    h = ln(x, tg_ref[...], tbt_ref[...])
    h = gelu(jnp.dot(h, tw_ref[...], preferred_element_type=jnp.float32)
             + tb_ref[...])
    x = x + h
    h = ln(x, cg_ref[...], cbt_ref[...])
    h = gelu(jnp.dot(h, cw_ref[...], preferred_element_type=jnp.float32)
             + cb_ref[...])
    x = x + h

    out_ref[...] = jnp.dot(sel_ref[...], x, preferred_element_type=jnp.float32)

  col = pl.BlockSpec((RR, 1), lambda i: (i, 0))
  full = lambda shp: pl.BlockSpec(shp, lambda i: (0, 0))
  return pl.pallas_call(
      body,
      grid=grid,
      in_specs=[
          pl.BlockSpec((RR, HID), lambda i: (i, 0)),
          col, col, col,
          full((1, TDIM)),
          full((RR, 1)), full((BB, RR)),
          full((DIMS, DIMS)), full((1, DIMS)),
          full((DIMS, DIMS)), full((1, DIMS)),
          full((1, DIMS)), full((1, DIMS)), full((1, DIMS)), full((1, DIMS)),
      ],
      out_specs=pl.BlockSpec((BB, DIMS), lambda i: (i, 0)),
      out_shape=jax.ShapeDtypeStruct((B, DIMS), jnp.float32),
  )(msg_rows, t_rows, tref_rows, mc_rows, freq_row,
    jnp.asarray(_SLOT_COL), jnp.asarray(_SEL),
    tW_t, tb, cW_t, cb, tg, tbeta, cg, cbeta)


def kernel(n_id, t_ref, msg_store, t_store, msg_count,
           token_gamma, token_beta, token_W, token_b,
           chan_gamma, chan_beta, chan_W, chan_b):
  nid2d = n_id.astype(jnp.int32).reshape(B // CH, CH)
  msg2d = msg_store.reshape(NUM_NODES, SIZE * HID)

  msg_g, t_g, mc_g = _sc_gather(nid2d, msg2d, t_store.reshape(-1), msg_count)

  msg_rows = msg_g.reshape(B * SIZE, HID)
  t_rows = jnp.transpose(t_g, (0, 2, 1)).reshape(B * SIZE, 1)
  tref_rows = jnp.repeat(t_ref, SIZE).reshape(B * SIZE, 1)
  mc_rows = jnp.repeat(mc_g.astype(jnp.float32), SIZE).reshape(B * SIZE, 1)

  freq_row = (1.0 / (10.0 ** jnp.linspace(0.0, 9.0, TDIM,
                                          dtype=jnp.float32))).reshape(1, TDIM)

  return _tc_mixer(msg_rows, t_rows, tref_rows, mc_rows, freq_row,
                   token_W.T, token_b.reshape(1, DIMS),
                   chan_W.T, chan_b.reshape(1, DIMS),
                   token_gamma.reshape(1, DIMS), token_beta.reshape(1, DIMS),
                   chan_gamma.reshape(1, DIMS), chan_beta.reshape(1, DIMS))
